# baseline (speedup 1.0000x reference)
import functools

import jax
import jax.numpy as jnp
from jax.experimental import pallas as pl
from jax.experimental.pallas import tpu as pltpu

B, T = 16, 2048
KSZ, NF, H = 3, 32, 128
DIST_MEAN, DIST_STD = 0.5, 0.29
S = T - KSZ + 1
TB = 186
NBLK = S // TB
TBP = TB + KSZ - 1
G4 = 4 * H


def _dot(a, b):
    return jax.lax.dot_general(a, b, (((1,), (0,)), ((), ())),
                               preferred_element_type=jnp.float32)


def _gates(g, c):
    ig = jax.nn.sigmoid(g[:, 0:H])
    fg = jax.nn.sigmoid(g[:, H:2 * H])
    gg = jnp.tanh(g[:, 2 * H:3 * H])
    og = jax.nn.sigmoid(g[:, 3 * H:])
    c = fg * c + ig * gg
    return og * jnp.tanh(c), c


def _body(geo_ref, wst_ref, wpcT_ref, bpc_ref, convW_ref, bconv_ref,
          wih0aT_ref, wd_ref, bih0_ref, bhh0_ref, wleft_ref,
          bih1_ref, bhh1_ref, lens_ref,
          out_ref, xg_ref, carry_ref):
    j = pl.program_id(0)
    t0 = j * TB

    geoW = geo_ref[pl.ds(t0 * B, TBP * B), :]

    a0 = wst_ref[0, 0]
    a1 = wst_ref[0, 1]
    d0 = wst_ref[1, 0] - a0
    d1 = wst_ref[1, 1] - a1
    wpcT = wpcT_ref[...]
    wpc_eff = jnp.concatenate(
        [wpcT[0:2], d0 * wpcT[2:3] + d1 * wpcT[3:4],
         jnp.zeros((1, 16), jnp.float32)], axis=0)
    bpc_eff = bpc_ref[...] + a0 * wpcT[2:3] + a1 * wpcT[3:4]

    proj = jnp.tanh(_dot(geoW, wpc_eff) + bpc_eff)

    acc = jnp.zeros((TB * B, NF), jnp.float32)
    for k in range(KSZ):
        acc = acc + _dot(proj[k * B:k * B + TB * B], convW_ref[k])
    pre = acc + bconv_ref[...]
    conv = jnp.where(pre > 0, pre, jnp.exp(jnp.minimum(pre, 0.0)) - 1.0)

    wd = wd_ref[...]
    wdm = jnp.concatenate(
        [jnp.zeros((3, G4), jnp.float32), wd / DIST_STD], axis=0)
    dgd = geoW[(KSZ - 1) * B:] - geoW[:TB * B]

    bias0 = bih0_ref[...] + bhh0_ref[...] - (DIST_MEAN / DIST_STD) * wd
    xg_ref[...] = _dot(conv, wih0aT_ref[...]) + _dot(dgd, wdm) + bias0

    @pl.when(j == 0)
    def _init():
        carry_ref[...] = jnp.zeros((4, B, H), jnp.float32)

    wleft = wleft_ref[...]
    bias1 = bih1_ref[...] + bhh1_ref[...]

    def step(tt, carry):
        h0, c0, h1, c1 = carry
        t = t0 + tt
        ga = _dot(jnp.concatenate([h0, h1], axis=1), wleft)
        h0n, c0n = _gates(ga[:, :G4] + xg_ref[pl.ds(tt * B, B), :], c0)
        h1n, c1n = _gates(ga[:, G4:] + bias1, c1)
        live = (t > 0).astype(jnp.float32)
        h1n = h1n * live
        c1n = c1n * live
        out_ref[jnp.maximum(t - 1, 0)] = h1n
        return h0n, c0n, h1n, c1n

    carry = (carry_ref[0], carry_ref[1], carry_ref[2], carry_ref[3])
    h0, c0, h1, c1 = jax.lax.fori_loop(0, TB, step, carry, unroll=93)
    carry_ref[0] = h0
    carry_ref[1] = c0
    carry_ref[2] = h1
    carry_ref[3] = c1

    @pl.when(j == NBLK - 1)
    def _tail():
        g1 = _dot(jnp.concatenate([h0, h1], axis=1), wleft)[:, G4:] + bias1
        h1n, _ = _gates(g1, c1)
        out_ref[S - 1] = h1n
        lensc = lens_ref[...] - (KSZ - 1)
        for m in range(NBLK):
            tids = m * TB + jax.lax.broadcasted_iota(
                jnp.int32, (TB, B, 1), 0)
            mask = (tids < lensc[None, :, :]).astype(jnp.float32)
            rows = out_ref[m * TB:(m + 1) * TB]
            out_ref[m * TB:(m + 1) * TB] = rows * mask


@functools.partial(jax.jit, static_argnums=())
def kernel(lngs, lats, states, dist_gap, lens, W_state, W_pc, b_pc, W_conv,
           b_conv, W_ih_l0, W_hh_l0, b_ih_l0, b_hh_l0, W_ih_l1, W_hh_l1,
           b_ih_l1, b_hh_l1):
    geo = jnp.stack(
        [lngs, lats, states.astype(jnp.float32), dist_gap], axis=-1)
    geo = jnp.transpose(geo, (1, 0, 2)).reshape(T * B, 4)

    wpcT = W_pc.T
    convW = jnp.transpose(W_conv, (2, 1, 0))
    wih0aT = W_ih_l0[:, :NF].T
    wd = W_ih_l0[:, NF][None, :]
    wleft = jnp.concatenate([
        jnp.concatenate([W_hh_l0.T, W_ih_l1.T], axis=1),
        jnp.concatenate([jnp.zeros((H, G4), jnp.float32), W_hh_l1.T],
                        axis=1)], axis=0)

    full = lambda shp: pl.BlockSpec(shp, lambda j: tuple(0 for _ in shp))
    out = pl.pallas_call(
        _body,
        grid=(NBLK,),
        in_specs=[
            full((T * B, 4)),
            full((2, 2)),
            full((4, 16)),
            full((1, 16)),
            full((KSZ, 16, NF)),
            full((1, NF)),
            full((NF, G4)),
            full((1, G4)),
            full((1, G4)),
            full((1, G4)),
            full((2 * H, 2 * G4)),
            full((1, G4)),
            full((1, G4)),
            full((B, 1)),
        ],
        out_specs=full((S, B, H)),
        out_shape=jax.ShapeDtypeStruct((S, B, H), jnp.float32),
        scratch_shapes=[
            pltpu.VMEM((TB * B, G4), jnp.float32),
            pltpu.VMEM((4, B, H), jnp.float32),
        ],
    )(geo, W_state, wpcT, b_pc[None, :], convW, b_conv[None, :],
      wih0aT, wd, b_ih_l0[None, :], b_hh_l0[None, :], wleft,
      b_ih_l1[None, :], b_hh_l1[None, :], lens[:, None])

    h_local = jnp.transpose(out, (1, 0, 2))
    return h_local, lens - (KSZ - 1)


# speedup vs baseline: 1.2036x; 1.2036x over previous
import functools

import jax
import jax.numpy as jnp
from jax.experimental import pallas as pl
from jax.experimental.pallas import tpu as pltpu

B, T = 16, 2048
KSZ, NF, H = 3, 32, 128
DIST_MEAN, DIST_STD = 0.5, 0.29
S = T - KSZ + 1
TB = 186
NBLK = S // TB
TBP = TB + KSZ - 1
G4 = 4 * H


def _dot(a, b):
    return jax.lax.dot_general(a, b, (((1,), (0,)), ((), ())),
                               preferred_element_type=jnp.float32)


def _gates(g, c):
    ig = jax.nn.sigmoid(g[:, 0:H])
    fg = jax.nn.sigmoid(g[:, H:2 * H])
    gg = jnp.tanh(g[:, 2 * H:3 * H])
    og = jax.nn.sigmoid(g[:, 3 * H:])
    c = fg * c + ig * gg
    return og * jnp.tanh(c), c


def _body(geo_ref, wst_ref, wpcT_ref, bpc_ref, convW_ref, bconv_ref,
          wih0aT_ref, wd_ref, bih0_ref, bhh0_ref, wleft_ref,
          whh1T_ref, bih1_ref, bhh1_ref, lens_ref,
          out_ref, xg_ref, carry_ref):
    j = pl.program_id(0)
    t0 = j * TB

    geoW = geo_ref[pl.ds(t0 * B, TBP * B), :]

    a0 = wst_ref[0, 0]
    a1 = wst_ref[0, 1]
    d0 = wst_ref[1, 0] - a0
    d1 = wst_ref[1, 1] - a1
    wpcT = wpcT_ref[...]
    wpc_eff = jnp.concatenate(
        [wpcT[0:2], d0 * wpcT[2:3] + d1 * wpcT[3:4],
         jnp.zeros((1, 16), jnp.float32)], axis=0)
    bpc_eff = bpc_ref[...] + a0 * wpcT[2:3] + a1 * wpcT[3:4]

    proj = jnp.tanh(_dot(geoW, wpc_eff) + bpc_eff)

    acc = jnp.zeros((TB * B, NF), jnp.float32)
    for k in range(KSZ):
        acc = acc + _dot(proj[k * B:k * B + TB * B], convW_ref[k])
    pre = acc + bconv_ref[...]
    conv = jnp.where(pre > 0, pre, jnp.exp(jnp.minimum(pre, 0.0)) - 1.0)

    wd = wd_ref[...]
    wdm = jnp.concatenate(
        [jnp.zeros((3, G4), jnp.float32), wd / DIST_STD], axis=0)
    dgd = geoW[(KSZ - 1) * B:] - geoW[:TB * B]

    bias0 = bih0_ref[...] + bhh0_ref[...] - (DIST_MEAN / DIST_STD) * wd
    xg_ref[...] = _dot(conv, wih0aT_ref[...]) + _dot(dgd, wdm) + bias0

    @pl.when(j == 0)
    def _init():
        carry_ref[...] = jnp.zeros((4, B, H), jnp.float32)

    wleft = wleft_ref[...]
    whh1T = whh1T_ref[...]
    bias1 = bih1_ref[...] + bhh1_ref[...]

    def step(tt, carry):
        h0, c0, h1, c1 = carry
        t = t0 + tt
        ga = _dot(h0, wleft)
        gb = _dot(h1, whh1T)
        h0n, c0n = _gates(ga[:, :G4] + xg_ref[pl.ds(tt * B, B), :], c0)
        h1n, c1n = _gates(ga[:, G4:] + gb + bias1, c1)
        live = (t > 0).astype(jnp.float32)
        h1n = h1n * live
        c1n = c1n * live
        out_ref[jnp.maximum(t - 1, 0)] = h1n
        return h0n, c0n, h1n, c1n

    carry = (carry_ref[0], carry_ref[1], carry_ref[2], carry_ref[3])
    h0, c0, h1, c1 = jax.lax.fori_loop(0, TB, step, carry, unroll=93)
    carry_ref[0] = h0
    carry_ref[1] = c0
    carry_ref[2] = h1
    carry_ref[3] = c1

    @pl.when(j == NBLK - 1)
    def _tail():
        g1 = _dot(h0, wleft)[:, G4:] + _dot(h1, whh1T) + bias1
        h1n, _ = _gates(g1, c1)
        out_ref[S - 1] = h1n
        lensc = lens_ref[...] - (KSZ - 1)
        for m in range(NBLK):
            tids = m * TB + jax.lax.broadcasted_iota(
                jnp.int32, (TB, B, 1), 0)
            mask = (tids < lensc[None, :, :]).astype(jnp.float32)
            rows = out_ref[m * TB:(m + 1) * TB]
            out_ref[m * TB:(m + 1) * TB] = rows * mask


@functools.partial(jax.jit, static_argnums=())
def kernel(lngs, lats, states, dist_gap, lens, W_state, W_pc, b_pc, W_conv,
           b_conv, W_ih_l0, W_hh_l0, b_ih_l0, b_hh_l0, W_ih_l1, W_hh_l1,
           b_ih_l1, b_hh_l1):
    geo = jnp.stack(
        [lngs, lats, states.astype(jnp.float32), dist_gap], axis=-1)
    geo = jnp.transpose(geo, (1, 0, 2)).reshape(T * B, 4)

    wpcT = W_pc.T
    convW = jnp.transpose(W_conv, (2, 1, 0))
    wih0aT = W_ih_l0[:, :NF].T
    wd = W_ih_l0[:, NF][None, :]
    wleft = jnp.concatenate([W_hh_l0, W_ih_l1], axis=0).T
    whh1T = W_hh_l1.T

    full = lambda shp: pl.BlockSpec(shp, lambda j: tuple(0 for _ in shp))
    out = pl.pallas_call(
        _body,
        grid=(NBLK,),
        in_specs=[
            full((T * B, 4)),
            full((2, 2)),
            full((4, 16)),
            full((1, 16)),
            full((KSZ, 16, NF)),
            full((1, NF)),
            full((NF, G4)),
            full((1, G4)),
            full((1, G4)),
            full((1, G4)),
            full((H, 2 * G4)),
            full((H, G4)),
            full((1, G4)),
            full((1, G4)),
            full((B, 1)),
        ],
        out_specs=full((S, B, H)),
        out_shape=jax.ShapeDtypeStruct((S, B, H), jnp.float32),
        scratch_shapes=[
            pltpu.VMEM((TB * B, G4), jnp.float32),
            pltpu.VMEM((4, B, H), jnp.float32),
        ],
    )(geo, W_state, wpcT, b_pc[None, :], convW, b_conv[None, :],
      wih0aT, wd, b_ih_l0[None, :], b_hh_l0[None, :], wleft,
      whh1T, b_ih_l1[None, :], b_hh_l1[None, :], lens[:, None])

    h_local = jnp.transpose(out, (1, 0, 2))
    return h_local, lens - (KSZ - 1)
